# baseline (speedup 1.0000x reference)
import jax
import jax.numpy as jnp
from jax.experimental import pallas as pl
from jax.experimental.pallas import tpu as pltpu

_MAP_SCALE = 0.1
_K = 5
_BBLK = 16


def _map_to_ball(x):
    sq = jnp.sum(x * x, axis=-1, keepdims=True)
    xn = x * jax.lax.rsqrt(jnp.maximum(sq, 1e-12)) * _MAP_SCALE
    n = jnp.maximum(jnp.sqrt(jnp.sum(xn * xn, axis=-1, keepdims=True)), 1e-15)
    v = jnp.tanh(n) * xn / n
    n2 = jnp.maximum(jnp.sqrt(jnp.sum(v * v, axis=-1, keepdims=True)), 1e-15)
    maxnorm = 1.0 - 1e-5
    return v * jnp.where(n2 > maxnorm, maxnorm / n2, 1.0)


def _distances(xsum, pk, seq):
    bblk = xsum.shape[0]
    qb = _map_to_ball(xsum * (1.0 / seq))
    kb = _map_to_ball(pk)
    x2 = jnp.sum(qb * qb, axis=1, keepdims=True)
    y2c = jnp.sum(kb * kb, axis=1, keepdims=True)
    dn_xy = (((1,), (1,)), ((), ()))
    xy = jax.lax.dot_general(qb, kb, dn_xy,
                             preferred_element_type=jnp.float32,
                             precision=jax.lax.Precision.HIGHEST)
    ones_col = jnp.ones((bblk, 1), jnp.float32)
    y2 = jax.lax.dot_general(ones_col, y2c, dn_xy,
                             preferred_element_type=jnp.float32,
                             precision=jax.lax.Precision.HIGHEST)
    a = 1.0 - 2.0 * xy + y2
    b = 1.0 - x2
    den = jnp.maximum(1.0 - 2.0 * xy + x2 * y2, 1e-15)
    n2 = a * a * x2 + b * b * y2 - 2.0 * a * b * xy
    dn = jnp.sqrt(jnp.maximum(n2, 0.0)) / den
    z = jnp.minimum(dn, 1.0 - 1e-5)
    dist = jnp.log((1.0 + z) / (1.0 - z))
    return dist, kb


def _topk_sel(sim, pool):
    bblk = sim.shape[0]
    iota = jax.lax.broadcasted_iota(jnp.int32, (bblk, pool), 1)
    active = jnp.full((bblk, pool), True)
    for _ in range(_K):
        m = jnp.max(jnp.where(active, sim, -jnp.inf), axis=1, keepdims=True)
        cand = jnp.where((sim == m) & active, iota, pool)
        pick = jnp.min(cand, axis=1, keepdims=True)
        active = active & (iota != pick)
    return jnp.logical_not(active), iota


def _body(x_ref, p2d_ref, pk_ref, out_ref, sim_ref, rs_ref, kn_ref, idx_ref):
    g = pl.program_id(0)
    bblk, seq, embed = x_ref.shape
    pool = pk_ref.shape[0]
    length = p2d_ref.shape[1] // embed

    out_ref[:, _K * length:, :] = x_ref[...]

    dist, kb = _distances(jnp.sum(x_ref[...], axis=1), pk_ref[...], seq)
    sim = -dist
    sim_ref[...] = sim

    sel, iota = _topk_sel(sim, pool)
    part = jnp.sum(jnp.where(sel, dist, 0.0))

    @pl.when(g == 0)
    def _():
        rs_ref[0, 0] = part

    @pl.when(g > 0)
    def _():
        rs_ref[0, 0] += part

    r0 = jax.lax.broadcasted_iota(jnp.int32, (pool, pool), 0)
    r1 = jax.lax.broadcasted_iota(jnp.int32, (pool, pool), 1)
    tri = (r0 <= r1).astype(jnp.float32)
    rank = jax.lax.dot_general(sel.astype(jnp.float32), tri,
                               (((1,), (0,)), ((), ())),
                               preferred_element_type=jnp.float32)
    dn_mm = (((1,), (0,)), ((), ()))
    for k in range(_K):
        cond = sel & (rank == float(k + 1))
        idxk = jnp.min(jnp.where(cond, iota, pool), axis=1, keepdims=True)
        idx_ref[:, k] = idxk[:, 0]
        oh = (iota == idxk).astype(jnp.float32)
        kn_ref[:, k, :] = jax.lax.dot_general(
            oh, kb, dn_mm, preferred_element_type=jnp.float32,
            precision=jax.lax.Precision.HIGHEST)
        chunk = jax.lax.dot_general(
            oh, p2d_ref[...], dn_mm,
            preferred_element_type=jnp.float32)
        for l in range(length):
            out_ref[:, k * length + l, :] = chunk[:, l * embed:(l + 1) * embed]


def kernel(x_embed, prompt, prompt_key):
    batch, seq, embed = x_embed.shape
    pool, length, _ = prompt.shape
    seq_out = _K * length + seq
    grid = (batch // _BBLK,)

    p2d = prompt.reshape(pool, length * embed)

    outs = pl.pallas_call(
        _body,
        grid=grid,
        in_specs=[
            pl.BlockSpec((_BBLK, seq, embed), lambda g: (g, 0, 0)),
            pl.BlockSpec((pool, length * embed), lambda g: (0, 0)),
            pl.BlockSpec((pool, embed), lambda g: (0, 0)),
        ],
        out_specs=[
            pl.BlockSpec((_BBLK, seq_out, embed), lambda g: (g, 0, 0)),
            pl.BlockSpec((_BBLK, pool), lambda g: (g, 0)),
            pl.BlockSpec((1, 1), lambda g: (0, 0),
                         memory_space=pltpu.SMEM),
            pl.BlockSpec((_BBLK, _K, embed), lambda g: (g, 0, 0)),
            pl.BlockSpec((_BBLK, _K), lambda g: (g, 0)),
        ],
        out_shape=[
            jax.ShapeDtypeStruct((batch, seq_out, embed), jnp.float32),
            jax.ShapeDtypeStruct((batch, pool), jnp.float32),
            jax.ShapeDtypeStruct((1, 1), jnp.float32),
            jax.ShapeDtypeStruct((batch, _K, embed), jnp.float32),
            jax.ShapeDtypeStruct((batch, _K), jnp.int32),
        ],
        compiler_params=pltpu.CompilerParams(
            dimension_semantics=("arbitrary",),
        ),
    )(x_embed, p2d, prompt_key)

    pe, sim, rs, kn, idx = outs
    return (pe, sim, rs[0, 0] * (1.0 / batch), kn, idx)
